# baseline (speedup 1.0000x reference)
import jax
import jax.numpy as jnp
from jax.experimental import pallas as pl
from jax.experimental.pallas import tpu as pltpu

_N = 10000
_S = 32
_D = 128
_U = 128
_B = 200


def _fused_body(x_ref, nb_ref, w_ref, b_ref, o_ref):
    s = jnp.sum(nb_ref[...], axis=1) + x_ref[...]
    h = s * (1.0 / (_S + 1))
    out = jnp.dot(h, w_ref[...], preferred_element_type=jnp.float32)
    o_ref[...] = jnp.maximum(out + b_ref[...], 0.0)


def kernel(x, neigh_x, kernel, bias):
    grid = _N // _B
    bias2 = bias.reshape(1, _U)
    return pl.pallas_call(
        _fused_body,
        grid=(grid,),
        in_specs=[
            pl.BlockSpec((_B, _D), lambda i: (i, 0)),
            pl.BlockSpec((_B, _S, _D), lambda i: (i, 0, 0)),
            pl.BlockSpec((_D, _U), lambda i: (0, 0)),
            pl.BlockSpec((1, _U), lambda i: (0, 0)),
        ],
        out_specs=pl.BlockSpec((_B, _U), lambda i: (i, 0)),
        out_shape=jax.ShapeDtypeStruct((_N, _U), jnp.float32),
        compiler_params=pltpu.CompilerParams(
            dimension_semantics=("arbitrary",),
        ),
    )(x, neigh_x, kernel, bias2)


# speedup vs baseline: 1.2538x; 1.2538x over previous
import jax
import jax.numpy as jnp
from jax.experimental import pallas as pl
from jax.experimental.pallas import tpu as pltpu

_N = 10000
_S = 32
_D = 128
_U = 128
_B = 400


def _fused_body(x_ref, nb_ref, w_ref, b_ref, o_ref):
    s = jnp.sum(nb_ref[...], axis=1) + x_ref[...]
    h = s * (1.0 / (_S + 1))
    out = jnp.dot(h, w_ref[...], preferred_element_type=jnp.float32)
    o_ref[...] = jnp.maximum(out + b_ref[...], 0.0)


def kernel(x, neigh_x, kernel, bias):
    grid = _N // _B
    bias2 = bias.reshape(1, _U)
    return pl.pallas_call(
        _fused_body,
        grid=(grid,),
        in_specs=[
            pl.BlockSpec((_B, _D), lambda i: (i, 0)),
            pl.BlockSpec((_B, _S, _D), lambda i: (i, 0, 0)),
            pl.BlockSpec((_D, _U), lambda i: (0, 0)),
            pl.BlockSpec((1, _U), lambda i: (0, 0)),
        ],
        out_specs=pl.BlockSpec((_B, _U), lambda i: (i, 0)),
        out_shape=jax.ShapeDtypeStruct((_N, _U), jnp.float32),
        compiler_params=pltpu.CompilerParams(
            dimension_semantics=("arbitrary",),
        ),
    )(x, neigh_x, kernel, bias2)
